# baseline (speedup 1.0000x reference)
import functools

import jax
import jax.numpy as jnp
from jax import lax
from jax.experimental import pallas as pl
from jax.experimental.pallas import tpu as pltpu
from jax.experimental.pallas import tpu_sc as plsc

N_DOF = 16777216
N_BND = 2097152
NW = 32
R = N_DOF // NW
CPY = 16384
NCH = R // CPY
WIN = 4096
PAD = WIN + 16


def _sc_body(q_hbm, si_hbm, sv_hbm, bnd_hbm, out_hbm,
             cpy0, cpy1, siw0, siw1, svw0, svw1, bnd_v,
             qsem, osem, lsem, bsem):
    cpy = (cpy0, cpy1)
    siw = (siw0, siw1)
    svw = (svw0, svw1)
    wid = lax.axis_index("s") * 2 + lax.axis_index("c")
    lo_t = wid * R
    lane = lax.iota(jnp.int32, 16)

    pltpu.async_copy(bnd_hbm.at[pl.ds(wid * NCH, NCH + 16)], bnd_v, bsem
                     ).wait()

    def extract(k):
        acc = jnp.zeros((16,), jnp.int32)
        for r in range(3):
            vec = bnd_v[pl.ds(r * 16, 16)]
            acc = acc + jnp.where(lane == (k - r * 16), vec, 0)
        return jnp.sum(acc)

    def win_start(c):
        e = extract(c)
        return pl.multiple_of(e - (e % 8), 8)

    def issue_in(c, par):
        pltpu.async_copy(
            q_hbm.at[pl.ds(lo_t + c * CPY, CPY)], cpy[par], qsem)
        ws = win_start(c)
        pltpu.async_copy(si_hbm.at[pl.ds(ws, WIN + 8)], siw[par], lsem)
        pltpu.async_copy(sv_hbm.at[pl.ds(ws, WIN)], svw[par], lsem)

    def wait_in(par):
        pltpu.make_async_copy(
            q_hbm.at[pl.ds(0, CPY)], cpy[par], qsem).wait()
        pltpu.make_async_copy(
            si_hbm.at[pl.ds(0, WIN + 8)], siw[par], lsem).wait()
        pltpu.make_async_copy(
            sv_hbm.at[pl.ds(0, WIN)], svw[par], lsem).wait()

    def wait_out(par):
        pltpu.make_async_copy(
            cpy[par], out_hbm.at[pl.ds(0, CPY)], osem).wait()

    def process(c, par):
        base_elem = lo_t + c * CPY

        @pl.loop(0, WIN // 16)
        def _(k):
            a = siw[par][pl.ds(k * 16, 16)]
            b = siw[par][pl.ds(k * 16 + 1, 16)]
            v = svw[par][pl.ds(k * 16, 16)]
            li = a - base_elem
            m = (a != b) & (li >= 0) & (li < CPY)
            plsc.store_scatter(cpy[par], (li,), v, mask=m)

        pltpu.async_copy(
            cpy[par], out_hbm.at[pl.ds(base_elem, CPY)], osem)

    issue_in(0, 0)
    issue_in(1, 1)

    def pair(p, carry):
        c0 = p * 2

        wait_in(0)
        process(c0, 0)

        wait_in(1)
        wait_out(0)

        @pl.when(c0 + 2 < NCH)
        def _():
            issue_in(c0 + 2, 0)

        process(c0 + 1, 1)
        wait_out(1)

        @pl.when(c0 + 3 < NCH)
        def _():
            issue_in(c0 + 3, 1)

        return carry

    lax.fori_loop(0, NCH // 2, pair, None)


@functools.cache
def _build():
    mesh = plsc.VectorSubcoreMesh(core_axis_name="c", subcore_axis_name="s")
    return pl.kernel(
        _sc_body,
        out_type=jax.ShapeDtypeStruct((N_DOF,), jnp.float32),
        mesh=mesh,
        compiler_params=pltpu.CompilerParams(needs_layout_passes=False),
        scratch_types=[
            pltpu.VMEM((CPY,), jnp.float32),
            pltpu.VMEM((CPY,), jnp.float32),
            pltpu.VMEM((WIN + 8,), jnp.int32),
            pltpu.VMEM((WIN + 8,), jnp.int32),
            pltpu.VMEM((WIN,), jnp.float32),
            pltpu.VMEM((WIN,), jnp.float32),
            pltpu.VMEM((NCH + 16,), jnp.int32),
            pltpu.SemaphoreType.DMA,
            pltpu.SemaphoreType.DMA,
            pltpu.SemaphoreType.DMA,
            pltpu.SemaphoreType.DMA,
        ],
    )


def kernel(q, _lambda, idx_b, xb_m, xb_c):
    idx = jnp.where(idx_b < 0, idx_b + N_DOF, idx_b)
    values = xb_m * _lambda + xb_c
    si, sv = lax.sort((idx, values), dimension=0, num_keys=1, is_stable=False)

    bounds = jnp.searchsorted(
        si, jnp.arange(N_DOF // CPY + 1, dtype=jnp.int32) * CPY
    ).astype(jnp.int32)
    bounds = jnp.pad(bounds, (0, 15))
    si_p = jnp.concatenate([si, jnp.full((PAD,), -1, jnp.int32)])
    sv_p = jnp.concatenate([sv, jnp.zeros((PAD,), jnp.float32)])

    return _build()(q, si_p, sv_p, bounds)
